# baseline (speedup 1.0000x reference)
import functools

import jax
import jax.numpy as jnp
from jax import lax
from jax.experimental import pallas as pl
from jax.experimental.pallas import tpu as pltpu, tpu_sc as plsc

EMBED_DIM = 32
BATCH = 16384
CHUNK = 128


@functools.cache
def _build(B, D):
    info = plsc.get_sparse_core_info()
    NC, NS = info.num_cores, info.num_subcores
    NW = NC * NS
    b_per_w = B // NW
    n_chunks = b_per_w // CHUNK
    mesh = plsc.VectorSubcoreMesh(core_axis_name="c", subcore_axis_name="s")

    @functools.partial(
        pl.kernel,
        out_type=jax.ShapeDtypeStruct((B, 2, D), jnp.float32),
        mesh=mesh,
        scratch_types=[
            pltpu.VMEM((n_chunks, CHUNK), jnp.int32),
            pltpu.VMEM((n_chunks, CHUNK), jnp.int32),
            pltpu.VMEM((b_per_w, D), jnp.float32),
            pltpu.VMEM((b_per_w, D), jnp.float32),
            pltpu.SemaphoreType.DMA,
            pltpu.SemaphoreType.DMA,
        ],
        compiler_params=pltpu.CompilerParams(use_tc_tiling_on_sc=False),
    )
    def k(emb_p, emb_n, xp, xn, out, idxp_v, idxn_v, rowsp_v, rowsn_v,
          semp, semn):
        wid = lax.axis_index("s") * NC + lax.axis_index("c")
        base = wid * b_per_w
        pltpu.sync_copy(xp.at[pl.ds(wid * n_chunks, n_chunks)], idxp_v)
        pltpu.sync_copy(xn.at[pl.ds(wid * n_chunks, n_chunks)], idxn_v)
        cps = []
        for j in range(n_chunks):
            cps.append(pltpu.async_copy(
                emb_p.at[idxp_v.at[j]],
                rowsp_v.at[pl.ds(j * CHUNK, CHUNK)], semp))
            cps.append(pltpu.async_copy(
                emb_n.at[idxn_v.at[j]],
                rowsn_v.at[pl.ds(j * CHUNK, CHUNK)], semn))
        for cp in cps:
            cp.wait()
        pltpu.sync_copy(rowsp_v, out.at[pl.ds(base, b_per_w), 0])
        pltpu.sync_copy(rowsn_v, out.at[pl.ds(base, b_per_w), 1])

    return k


def kernel(x, emb_proton, emb_neutron):
    B, D = BATCH, EMBED_DIM
    xi = x.astype(jnp.int32)
    xp = xi[:, 0].reshape(B // CHUNK, CHUNK)
    xn = xi[:, 1].reshape(B // CHUNK, CHUNK)
    return _build(B, D)(emb_proton, emb_neutron, xp, xn)


# speedup vs baseline: 3.4152x; 3.4152x over previous
import functools

import jax
import jax.numpy as jnp
from jax import lax
from jax.experimental import pallas as pl
from jax.experimental.pallas import tpu as pltpu, tpu_sc as plsc

EMBED_DIM = 32
BATCH = 16384
OUT_CHUNK = 8192


@functools.cache
def _build(B, D):
    info = plsc.get_sparse_core_info()
    NC, NS, L = info.num_cores, info.num_subcores, info.num_lanes
    d_per_tile = D // NS
    n_out_chunks = B // OUT_CHUNK
    mesh = plsc.VectorSubcoreMesh(core_axis_name="c", subcore_axis_name="s")

    @functools.partial(
        pl.kernel,
        out_type=jax.ShapeDtypeStruct((2, D, B), jnp.float32),
        mesh=mesh,
        scratch_types=[
            pltpu.VMEM((100000,), jnp.float32),
            pltpu.VMEM((B,), jnp.int32),
            pltpu.VMEM((OUT_CHUNK,), jnp.float32),
        ],
        compiler_params=pltpu.CompilerParams(
            use_tc_tiling_on_sc=True, needs_layout_passes=False),
    )
    def k(tp, tn, xp, xn, out, row_v, idx_v, out_v):
        cid = lax.axis_index("c")
        sid = lax.axis_index("s")

        def run(tT, xk, kk):
            pltpu.sync_copy(xk, idx_v)
            for t in range(d_per_tile):
                d = sid * d_per_tile + t
                pltpu.sync_copy(tT.at[d], row_v)
                for c in range(n_out_chunks):
                    @pl.loop(0, OUT_CHUNK // L)
                    def _(i):
                        idx = idx_v[pl.ds(c * OUT_CHUNK + i * L, L)]
                        out_v[pl.ds(i * L, L)] = plsc.load_gather(
                            row_v, [idx])
                    pltpu.sync_copy(
                        out_v, out.at[kk, d, pl.ds(c * OUT_CHUNK, OUT_CHUNK)])

        @pl.when(cid == 0)
        def _():
            run(tp, xp, 0)

        @pl.when(cid == 1)
        def _():
            run(tn, xn, 1)

    return k


def kernel(x, emb_proton, emb_neutron):
    B, D = BATCH, EMBED_DIM
    xi = x.astype(jnp.int32)
    xp = xi[:, 0]
    xn = xi[:, 1]
    out = _build(B, D)(emb_proton.T, emb_neutron.T, xp, xn)
    return out.transpose(2, 0, 1)
